# baseline (speedup 1.0000x reference)
import math

import jax
import jax.numpy as jnp
from jax.experimental import pallas as pl
from jax.experimental.pallas import tpu as pltpu

_BN_SCALE = 1.0 / math.sqrt(1.0 + 1e-5)
_ROW_BLOCK1 = 400
_ROW_BLOCK2 = 1000
_VMEM_LIMIT = 56 * 1024 * 1024


def kernel(x, adj, enc_w1, enc_b1, enc_w2, enc_b2, gc1_w, gc2_w, gc3_w,
           att_w, dec_w1, dec_b1, dec_w2, dec_b2):
    n, d = x.shape
    gh1 = gc1_w.shape[1]
    gh2 = gc2_w.shape[1]
    fh0 = enc_w1.shape[1]
    fh2 = enc_w2.shape[1]
    blk1 = _ROW_BLOCK1
    nb1 = n // blk1
    blk2 = _ROW_BLOCK2
    nb2 = n // blk2

    g23 = jnp.concatenate([gc2_w, gc3_w], axis=1)

    def _pass1_kernel(adj_ref, x_ref, gc1_ref, g23_ref, adj16_ref, t_ref,
                      s_ref):
        @pl.when(pl.program_id(0) == 0)
        def _():
            s_ref[...] = jnp.dot(
                x_ref[...], gc1_ref[...],
                preferred_element_type=jnp.float32).astype(jnp.bfloat16)

        a16 = adj_ref[...].astype(jnp.bfloat16)
        adj16_ref[...] = a16
        h1 = jnp.maximum(
            jnp.dot(a16, s_ref[...], preferred_element_type=jnp.float32),
            0.0)
        t_ref[...] = jnp.dot(
            h1, g23_ref[...],
            preferred_element_type=jnp.float32).astype(jnp.bfloat16)

    adj16, t = pl.pallas_call(
        _pass1_kernel,
        grid=(nb1,),
        in_specs=[
            pl.BlockSpec((blk1, n), lambda i: (i, 0)),
            pl.BlockSpec((n, d), lambda i: (0, 0)),
            pl.BlockSpec((d, gh1), lambda i: (0, 0)),
            pl.BlockSpec((gh1, 2 * gh2), lambda i: (0, 0)),
        ],
        out_specs=[
            pl.BlockSpec((blk1, n), lambda i: (i, 0)),
            pl.BlockSpec((blk1, 2 * gh2), lambda i: (i, 0)),
        ],
        out_shape=(
            jax.ShapeDtypeStruct((n, n), jnp.bfloat16),
            jax.ShapeDtypeStruct((n, 2 * gh2), jnp.bfloat16),
        ),
        scratch_shapes=[pltpu.VMEM((n, gh1), jnp.bfloat16)],
        compiler_params=pltpu.CompilerParams(vmem_limit_bytes=_VMEM_LIMIT),
    )(adj, x, gc1_w, g23)

    eb1 = enc_b1.reshape(1, fh0)
    eb2 = enc_b2.reshape(1, fh2)
    db1 = dec_b1.reshape(1, fh0)
    db2 = dec_b2.reshape(1, d)
    att = att_w.reshape(1, gh2)

    def _pass2_kernel(a16_ref, t_ref, x_ref, ew1_ref, eb1_ref, ew2_ref,
                      eb2_ref, att_ref, dw1_ref, db1_ref, dw2_ref, db2_ref,
                      z_ref, mu_ref, lv_ref, df_ref):
        ml = jnp.dot(a16_ref[...], t_ref[...],
                     preferred_element_type=jnp.float32)
        mu = ml[:, :gh2]
        lv = ml[:, gh2:]
        h = jnp.maximum(
            (jnp.dot(x_ref[...], ew1_ref[...],
                     preferred_element_type=jnp.float32)
             + eb1_ref[...]) * _BN_SCALE, 0.0)
        feat = jnp.maximum(
            (jnp.dot(h, ew2_ref[...], preferred_element_type=jnp.float32)
             + eb2_ref[...]) * _BN_SCALE, 0.0)
        a = att_ref[...]
        wg = jnp.sum(mu * a, axis=1, keepdims=True)
        wf = jnp.sum(feat * a, axis=1, keepdims=True)
        m = jnp.maximum(wg, wf)
        eg = jnp.exp(wg - m)
        ef = jnp.exp(wf - m)
        z = (eg * mu + ef * feat) / (eg + ef)
        dh = jnp.maximum(
            (jnp.dot(z, dw1_ref[...], preferred_element_type=jnp.float32)
             + db1_ref[...]) * _BN_SCALE, 0.0)
        df_ref[...] = (jnp.dot(dh, dw2_ref[...],
                               preferred_element_type=jnp.float32)
                       + db2_ref[...])
        z_ref[...] = z
        mu_ref[...] = mu
        lv_ref[...] = lv

    z, mu, lv, df = pl.pallas_call(
        _pass2_kernel,
        grid=(nb2,),
        in_specs=[
            pl.BlockSpec((blk2, n), lambda i: (i, 0)),
            pl.BlockSpec((n, 2 * gh2), lambda i: (0, 0)),
            pl.BlockSpec((blk2, d), lambda i: (i, 0)),
            pl.BlockSpec((d, fh0), lambda i: (0, 0)),
            pl.BlockSpec((1, fh0), lambda i: (0, 0)),
            pl.BlockSpec((fh0, fh2), lambda i: (0, 0)),
            pl.BlockSpec((1, fh2), lambda i: (0, 0)),
            pl.BlockSpec((1, gh2), lambda i: (0, 0)),
            pl.BlockSpec((fh2, fh0), lambda i: (0, 0)),
            pl.BlockSpec((1, fh0), lambda i: (0, 0)),
            pl.BlockSpec((fh0, d), lambda i: (0, 0)),
            pl.BlockSpec((1, d), lambda i: (0, 0)),
        ],
        out_specs=[
            pl.BlockSpec((blk2, gh2), lambda i: (i, 0)),
            pl.BlockSpec((blk2, gh2), lambda i: (i, 0)),
            pl.BlockSpec((blk2, gh2), lambda i: (i, 0)),
            pl.BlockSpec((blk2, d), lambda i: (i, 0)),
        ],
        out_shape=(
            jax.ShapeDtypeStruct((n, gh2), jnp.float32),
            jax.ShapeDtypeStruct((n, gh2), jnp.float32),
            jax.ShapeDtypeStruct((n, gh2), jnp.float32),
            jax.ShapeDtypeStruct((n, d), jnp.float32),
        ),
        compiler_params=pltpu.CompilerParams(vmem_limit_bytes=_VMEM_LIMIT),
    )(adj16, t, x, enc_w1, eb1, enc_w2, eb2, att, dec_w1, db1, dec_w2, db2)
    return (z, mu, lv, df)


# speedup vs baseline: 1.1150x; 1.1150x over previous
import math

import jax
import jax.numpy as jnp
from jax.experimental import pallas as pl
from jax.experimental.pallas import tpu as pltpu

_BN_SCALE = 1.0 / math.sqrt(1.0 + 1e-5)
_ROW_BLOCK = 400
_VMEM_LIMIT = 56 * 1024 * 1024


def kernel(x, adj, enc_w1, enc_b1, enc_w2, enc_b2, gc1_w, gc2_w, gc3_w,
           att_w, dec_w1, dec_b1, dec_w2, dec_b2):
    n, d = x.shape
    gh1 = gc1_w.shape[1]
    gh2 = gc2_w.shape[1]
    fh0 = enc_w1.shape[1]
    fh2 = enc_w2.shape[1]
    blk = _ROW_BLOCK
    nb = n // blk

    g23 = jnp.concatenate([gc2_w, gc3_w], axis=1)
    eb1 = enc_b1.reshape(1, fh0)
    eb2 = enc_b2.reshape(1, fh2)
    db1 = dec_b1.reshape(1, fh0)
    db2 = dec_b2.reshape(1, d)
    att = att_w.reshape(1, gh2)

    def _fused_kernel(adj_ref, x_ref, gc1_ref, g23_ref, ew1_ref, eb1_ref,
                      ew2_ref, eb2_ref, att_ref, dw1_ref, db1_ref, dw2_ref,
                      db2_ref, z_ref, mu_ref, lv_ref, df_ref, s_ref, t_ref):
        i = pl.program_id(0)

        @pl.when(i == 0)
        def _():
            s_ref[...] = jnp.dot(x_ref[...], gc1_ref[...],
                                 preferred_element_type=jnp.float32)

        @pl.when(i < nb)
        def _():
            h1 = jnp.maximum(
                jnp.dot(adj_ref[...], s_ref[...],
                        preferred_element_type=jnp.float32), 0.0)
            t_ref[pl.ds(i * blk, blk), :] = jnp.dot(
                h1, g23_ref[...], preferred_element_type=jnp.float32)

        @pl.when(i >= nb)
        def _():
            j = i - nb
            ml = jnp.dot(adj_ref[...], t_ref[...],
                         preferred_element_type=jnp.float32)
            mu = ml[:, :gh2]
            lv = ml[:, gh2:]
            xb = x_ref[pl.ds(j * blk, blk), :]
            h = jnp.maximum(
                (jnp.dot(xb, ew1_ref[...], preferred_element_type=jnp.float32)
                 + eb1_ref[...]) * _BN_SCALE, 0.0)
            feat = jnp.maximum(
                (jnp.dot(h, ew2_ref[...], preferred_element_type=jnp.float32)
                 + eb2_ref[...]) * _BN_SCALE, 0.0)
            a = att_ref[...]
            wg = jnp.sum(mu * a, axis=1, keepdims=True)
            wf = jnp.sum(feat * a, axis=1, keepdims=True)
            m = jnp.maximum(wg, wf)
            eg = jnp.exp(wg - m)
            ef = jnp.exp(wf - m)
            z = (eg * mu + ef * feat) / (eg + ef)
            dh = jnp.maximum(
                (jnp.dot(z, dw1_ref[...], preferred_element_type=jnp.float32)
                 + db1_ref[...]) * _BN_SCALE, 0.0)
            df_ref[...] = (jnp.dot(dh, dw2_ref[...],
                                   preferred_element_type=jnp.float32)
                           + db2_ref[...])
            z_ref[...] = z
            mu_ref[...] = mu
            lv_ref[...] = lv

    out_map = lambda i: (jnp.maximum(i - nb, 0), 0)
    z, mu, lv, df = pl.pallas_call(
        _fused_kernel,
        grid=(2 * nb,),
        in_specs=[
            pl.BlockSpec((blk, n), lambda i: (i % nb, 0)),
            pl.BlockSpec((n, d), lambda i: (0, 0)),
            pl.BlockSpec((d, gh1), lambda i: (0, 0)),
            pl.BlockSpec((gh1, 2 * gh2), lambda i: (0, 0)),
            pl.BlockSpec((d, fh0), lambda i: (0, 0)),
            pl.BlockSpec((1, fh0), lambda i: (0, 0)),
            pl.BlockSpec((fh0, fh2), lambda i: (0, 0)),
            pl.BlockSpec((1, fh2), lambda i: (0, 0)),
            pl.BlockSpec((1, gh2), lambda i: (0, 0)),
            pl.BlockSpec((fh2, fh0), lambda i: (0, 0)),
            pl.BlockSpec((1, fh0), lambda i: (0, 0)),
            pl.BlockSpec((fh0, d), lambda i: (0, 0)),
            pl.BlockSpec((1, d), lambda i: (0, 0)),
        ],
        out_specs=[
            pl.BlockSpec((blk, gh2), out_map),
            pl.BlockSpec((blk, gh2), out_map),
            pl.BlockSpec((blk, gh2), out_map),
            pl.BlockSpec((blk, d), out_map),
        ],
        out_shape=(
            jax.ShapeDtypeStruct((n, gh2), jnp.float32),
            jax.ShapeDtypeStruct((n, gh2), jnp.float32),
            jax.ShapeDtypeStruct((n, gh2), jnp.float32),
            jax.ShapeDtypeStruct((n, d), jnp.float32),
        ),
        scratch_shapes=[
            pltpu.VMEM((n, gh1), jnp.float32),
            pltpu.VMEM((n, 2 * gh2), jnp.float32),
        ],
        compiler_params=pltpu.CompilerParams(vmem_limit_bytes=_VMEM_LIMIT),
    )(adj, x, gc1_w, g23, enc_w1, eb1, enc_w2, eb2, att, dec_w1, db1,
      dec_w2, db2)
    return (z, mu, lv, df)


# speedup vs baseline: 1.1182x; 1.0029x over previous
import math

import jax
import jax.numpy as jnp
from jax.experimental import pallas as pl
from jax.experimental.pallas import tpu as pltpu

_BN_SCALE = 1.0 / math.sqrt(1.0 + 1e-5)
_ROW_BLOCK = 400
_VMEM_LIMIT = 56 * 1024 * 1024


def kernel(x, adj, enc_w1, enc_b1, enc_w2, enc_b2, gc1_w, gc2_w, gc3_w,
           att_w, dec_w1, dec_b1, dec_w2, dec_b2):
    n, d = x.shape
    gh1 = gc1_w.shape[1]
    gh2 = gc2_w.shape[1]
    fh0 = enc_w1.shape[1]
    fh2 = enc_w2.shape[1]
    blk = _ROW_BLOCK
    nb = n // blk

    g23 = jnp.concatenate([gc2_w, gc3_w], axis=1)
    eb1 = enc_b1.reshape(1, fh0)
    eb2 = enc_b2.reshape(1, fh2)
    db1 = dec_b1.reshape(1, fh0)
    db2 = dec_b2.reshape(1, d)
    att = att_w.reshape(1, gh2)

    def _fused_kernel(adj_ref, x_ref, gc1_ref, g23_ref, ew1_ref, eb1_ref,
                      ew2_ref, eb2_ref, att_ref, dw1_ref, db1_ref, dw2_ref,
                      db2_ref, z_ref, mu_ref, lv_ref, df_ref, s_ref, t_ref):
        i = pl.program_id(0)

        @pl.when(i == 0)
        def _():
            s_ref[...] = jnp.dot(x_ref[...], gc1_ref[...],
                                 preferred_element_type=jnp.float32)

        @pl.when(i < nb)
        def _():
            h1 = jnp.maximum(
                jnp.dot(adj_ref[:, :5120], s_ref[:5120, :],
                        preferred_element_type=jnp.float32)
                + jnp.dot(adj_ref[:, 5120:], s_ref[5120:, :],
                          preferred_element_type=jnp.float32), 0.0)
            t_ref[pl.ds(i * blk, blk), :] = jnp.dot(
                h1, g23_ref[...], preferred_element_type=jnp.float32)

        @pl.when(i >= nb)
        def _():
            j = i - nb
            ml = (jnp.dot(adj_ref[:, :5120], t_ref[:5120, :],
                          preferred_element_type=jnp.float32)
                  + jnp.dot(adj_ref[:, 5120:], t_ref[5120:, :],
                            preferred_element_type=jnp.float32))
            mu = ml[:, :gh2]
            lv = ml[:, gh2:]
            xb = x_ref[pl.ds(j * blk, blk), :]
            h = jnp.maximum(
                (jnp.dot(xb, ew1_ref[...], preferred_element_type=jnp.float32)
                 + eb1_ref[...]) * _BN_SCALE, 0.0)
            feat = jnp.maximum(
                (jnp.dot(h, ew2_ref[...], preferred_element_type=jnp.float32)
                 + eb2_ref[...]) * _BN_SCALE, 0.0)
            a = att_ref[...]
            wg = jnp.sum(mu * a, axis=1, keepdims=True)
            wf = jnp.sum(feat * a, axis=1, keepdims=True)
            m = jnp.maximum(wg, wf)
            eg = jnp.exp(wg - m)
            ef = jnp.exp(wf - m)
            z = (eg * mu + ef * feat) / (eg + ef)
            dh = jnp.maximum(
                (jnp.dot(z, dw1_ref[...], preferred_element_type=jnp.float32)
                 + db1_ref[...]) * _BN_SCALE, 0.0)
            df_ref[...] = (jnp.dot(dh, dw2_ref[...],
                                   preferred_element_type=jnp.float32)
                           + db2_ref[...])
            z_ref[...] = z
            mu_ref[...] = mu
            lv_ref[...] = lv

    out_map = lambda i: (jnp.maximum(i - nb, 0), 0)
    z, mu, lv, df = pl.pallas_call(
        _fused_kernel,
        grid=(2 * nb,),
        in_specs=[
            pl.BlockSpec((blk, n), lambda i: (i % nb, 0)),
            pl.BlockSpec((n, d), lambda i: (0, 0)),
            pl.BlockSpec((d, gh1), lambda i: (0, 0)),
            pl.BlockSpec((gh1, 2 * gh2), lambda i: (0, 0)),
            pl.BlockSpec((d, fh0), lambda i: (0, 0)),
            pl.BlockSpec((1, fh0), lambda i: (0, 0)),
            pl.BlockSpec((fh0, fh2), lambda i: (0, 0)),
            pl.BlockSpec((1, fh2), lambda i: (0, 0)),
            pl.BlockSpec((1, gh2), lambda i: (0, 0)),
            pl.BlockSpec((fh2, fh0), lambda i: (0, 0)),
            pl.BlockSpec((1, fh0), lambda i: (0, 0)),
            pl.BlockSpec((fh0, d), lambda i: (0, 0)),
            pl.BlockSpec((1, d), lambda i: (0, 0)),
        ],
        out_specs=[
            pl.BlockSpec((blk, gh2), out_map),
            pl.BlockSpec((blk, gh2), out_map),
            pl.BlockSpec((blk, gh2), out_map),
            pl.BlockSpec((blk, d), out_map),
        ],
        out_shape=(
            jax.ShapeDtypeStruct((n, gh2), jnp.float32),
            jax.ShapeDtypeStruct((n, gh2), jnp.float32),
            jax.ShapeDtypeStruct((n, gh2), jnp.float32),
            jax.ShapeDtypeStruct((n, d), jnp.float32),
        ),
        scratch_shapes=[
            pltpu.VMEM((n, gh1), jnp.float32),
            pltpu.VMEM((n, 2 * gh2), jnp.float32),
        ],
        compiler_params=pltpu.CompilerParams(vmem_limit_bytes=_VMEM_LIMIT),
    )(adj, x, gc1_w, g23, enc_w1, eb1, enc_w2, eb2, att, dec_w1, db1,
      dec_w2, db2)
    return (z, mu, lv, df)
